# baseline (speedup 1.0000x reference)
import jax
import jax.numpy as jnp
from jax import lax
from jax.experimental import pallas as pl
from jax.experimental.pallas import tpu as pltpu
from jax.experimental.pallas import tpu_sc as plsc

H = 16
DT = 0.01

NW1, NB1, NW2, NB2 = 0, 1, 2, 18
CW1, CB1, CW2, CB2 = 19, 20, 21, 37
LW1, LB1, LW2, LB2 = 38, 39, 40, 56
P1A, P1B, P1C, PB1, P2, PB2 = 57, 73, 89, 105, 106, 122
Q1A, Q1B, QB1, Q2, QB2 = 123, 139, 155, 156, 172
D1, DB1, D2, DB2, D3 = 173, 189, 190, 206, 207
C1, CDB1, C2, CDB2, C3 = 208, 224, 225, 241, 242
L1, LDB1, L2, LDB2, L3 = 243, 259, 260, 276, 277
MISC = 278
NROWS = 279


def _bf16r(v):
    u = lax.bitcast_convert_type(v, jnp.uint32)
    r = (u + jnp.uint32(0x7FFF) + ((u >> jnp.uint32(16)) & jnp.uint32(1))) \
        & jnp.uint32(0xFFFF0000)
    return lax.bitcast_convert_type(r, jnp.float32)


def _forward(p_v):
    zero = jnp.zeros((H,), jnp.float32)

    def mm(specs):
        accs = [[init, zero] for (init, _) in specs]
        rcache = {}
        for (_, pairs) in specs:
            for (av, _wb) in pairs:
                if id(av) not in rcache:
                    rcache[id(av)] = _bf16r(av)
        for k in range(H):
            wcache = {}
            scache = {}
            for j, (_, pairs) in enumerate(specs):
                for (av, wb) in pairs:
                    if wb not in wcache:
                        wcache[wb] = p_v[wb + k, :]
                    ai = id(av)
                    if ai not in scache:
                        scache[ai] = rcache[ai][k]
                    accs[j][k % 2] = accs[j][k % 2] + scache[ai] * wcache[wb]
        return [a[0] + a[1] for a in accs]

    misc = p_v[MISC, :]
    x = [misc[3], misc[4], misc[5]]
    nval = [misc[6], misc[7], misc[8]]

    nw1, nb1 = p_v[NW1, :], p_v[NB1, :]
    an = [jnp.maximum(nval[v] * nw1 + nb1, 0.0) for v in range(3)]

    cw1, cb1 = p_v[CW1, :], p_v[CB1, :]
    lw1, lb1 = p_v[LW1, :], p_v[LB1, :]
    ew1 = [cw1, lw1, cw1]
    eb1 = [cb1, lb1, cb1]
    ae, tae = [], []
    for i in range(3):
        u = x[i] * ew1[i] + eb1[i]
        ae.append(jnp.maximum(u, 0.0))
        tae.append(jnp.where(u > 0, ew1[i], 0.0))

    nb2, cb2, lb2 = p_v[NB2, :], p_v[CB2, :], p_v[LB2, :]
    ew2 = [CW2, LW2, CW2]
    eb2 = [cb2, lb2, cb2]
    res = mm(
        [(nb2, [(an[v], NW2)]) for v in range(3)]
        + [(eb2[i], [(ae[i], ew2[i])]) for i in range(3)]
        + [(zero, [(tae[i], ew2[i])]) for i in range(3)]
    )
    hn0 = res[0:3]
    he0 = res[3:6]
    t0 = res[6:9]

    pb1 = p_v[PB1, :]
    snd = (0, 1, 0)
    rcv = (1, 2, 2)
    res = mm(
        [(pb1, [(he0[i], P1A), (hn0[snd[i]], P1B), (hn0[rcv[i]], P1C)])
         for i in range(3)]
        + [(zero, [(t0[i], P1A)]) for i in range(3)]
    )
    wp, twp = [], []
    for i in range(3):
        u = res[i]
        wp.append(jnp.maximum(u, 0.0))
        twp.append(jnp.where(u > 0, res[3 + i], 0.0))

    pb2 = p_v[PB2, :]
    res = mm(
        [(he0[i] + pb2, [(wp[i], P2)]) for i in range(3)]
        + [(t0[i], [(twp[i], P2)]) for i in range(3)]
    )
    he1 = res[0:3]
    t1 = res[3:6]

    agg = [he1[0], he1[1] + he1[2]]

    qb1 = p_v[QB1, :]
    res = mm([(qb1, [(hn0[1 + v], Q1A), (agg[v], Q1B)]) for v in range(2)])
    nwr = [jnp.maximum(res[v], 0.0) for v in range(2)]

    qb2 = p_v[QB2, :]
    hn1 = mm([(hn0[1 + v] + qb2, [(nwr[v], Q2)]) for v in range(2)])

    db1 = p_v[DB1, :]
    cdb1, ldb1 = p_v[CDB1, :], p_v[LDB1, :]
    e1w = [C1, L1, C1]
    e1b = [cdb1, ldb1, cdb1]
    res = mm(
        [(db1, [(hn1[v], D1)]) for v in range(2)]
        + [(e1b[i], [(he1[i], e1w[i])]) for i in range(3)]
        + [(zero, [(t1[i], e1w[i])]) for i in range(3)]
    )
    z1 = [jnp.maximum(res[v], 0.0) for v in range(2)]
    y1, ty1 = [], []
    for i in range(3):
        u = res[2 + i]
        y1.append(jnp.maximum(u, 0.0))
        ty1.append(jnp.where(u > 0, res[5 + i], 0.0))

    db2 = p_v[DB2, :]
    cdb2, ldb2 = p_v[CDB2, :], p_v[LDB2, :]
    e2w = [C2, L2, C2]
    e2b = [cdb2, ldb2, cdb2]
    res = mm(
        [(db2, [(z1[v], D2)]) for v in range(2)]
        + [(e2b[i], [(y1[i], e2w[i])]) for i in range(3)]
        + [(zero, [(ty1[i], e2w[i])]) for i in range(3)]
    )
    z2 = [jnp.maximum(res[v], 0.0) for v in range(2)]
    y2 = [jnp.maximum(res[2 + i], 0.0) for i in range(3)]
    ty2 = [jnp.where(res[2 + i] > 0, res[5 + i], 0.0) for i in range(3)]

    d3 = p_v[D3, :]
    e3w = [p_v[C3, :], p_v[L3, :], p_v[C3, :]]
    lane = lax.iota(jnp.int32, 16)
    dnums = lax.GatherDimensionNumbers(
        offset_dims=(), collapsed_slice_dims=(0,), start_index_map=(0,))

    z2r = [_bf16r(z2[v]) for v in range(2)]
    y2r = [_bf16r(y2[i]) for i in range(3)]
    ty2r = [_bf16r(ty2[i]) for i in range(3)]
    prods = [
        z2r[0] * d3, z2r[1] * d3,
        ty2r[0] * e3w[0], ty2r[1] * e3w[1], ty2r[2] * e3w[2],
        y2r[0] * e3w[0] + y2r[1] * e3w[1] + y2r[2] * e3w[2],
    ]
    for sh in (1, 2, 4, 8):
        idx = (lane ^ sh)[:, None]
        prods = [
            v + lax.gather(v, idx, dnums, slice_sizes=(1,),
                           mode=lax.GatherScatterMode.PROMISE_IN_BOUNDS)
            for v in prods
        ]
    db3n = misc[0]
    decn = [prods[0] + db3n, prods[1] + db3n]
    dh = prods[2:5]
    ham = prods[5] + misc[1] + misc[2] + misc[1]

    c2 = misc[11]
    nx0 = x[0] + DT * dh[1]
    nx1 = x[1] + DT * (-dh[0] + dh[2])
    nx2 = x[2] + DT * (-dh[1] - c2)
    return (
        jnp.where(lane == 1, decn[0], 0.0)
        + jnp.where(lane == 2, decn[1], 0.0)
        + jnp.where(lane == 3, nx0, 0.0)
        + jnp.where(lane == 4, nx1, 0.0)
        + jnp.where(lane == 5, nx2, 0.0)
        + jnp.where(lane == 6, ham, 0.0)
    )


def _body(p_hbm, out_hbm, p_v, o_v):
    c = lax.axis_index("c")
    s = lax.axis_index("s")

    @pl.when(jnp.logical_and(c == 0, s == 0))
    def _():
        pltpu.sync_copy(p_hbm, p_v)
        o_v[...] = _forward(p_v)
        pltpu.sync_copy(o_v, out_hbm)


def _pack(nodes, edges, control, enc_node, enc_edgeC, enc_edgeL, proc_edge, proc_node, dec_node, dec_edgeC, dec_edgeL):
    def r(w):
        return _bf16r(w).reshape(-1)

    parts = []
    for p in (enc_node, enc_edgeC, enc_edgeL):
        parts.extend([p[0].reshape(-1), p[1], r(p[2]), p[3]])
    for p in (proc_edge, proc_node):
        parts.extend([r(p[0]), p[1], r(p[2]), p[3]])
    for p in (dec_node, dec_edgeC, dec_edgeL):
        parts.extend([r(p[0]), p[1], r(p[2]), p[3], r(p[4])])
    parts.extend([
        dec_node[5], dec_edgeC[5], dec_edgeL[5],
        edges.reshape(-1), nodes.reshape(-1), control,
        jnp.zeros((4,), jnp.float32),
    ])
    return jnp.concatenate(parts).reshape(NROWS, H)


def kernel(nodes, edges, control, enc_node, enc_edgeC, enc_edgeL, proc_edge, proc_node, dec_node, dec_edgeC, dec_edgeL):
    packed = _pack(nodes, edges, control, enc_node, enc_edgeC, enc_edgeL,
                   proc_edge, proc_node, dec_node, dec_edgeC, dec_edgeL)

    mesh = plsc.VectorSubcoreMesh(core_axis_name="c", subcore_axis_name="s")
    run = pl.kernel(
        _body,
        out_type=jax.ShapeDtypeStruct((H,), jnp.float32),
        mesh=mesh,
        scratch_types=[
            pltpu.VMEM((NROWS, H), jnp.float32),
            pltpu.VMEM((H,), jnp.float32),
        ],
    )
    out = run(packed)

    next_nodes = out[0:3].reshape(3, 1)
    next_edges = out[3:6].reshape(3, 1)
    next_globals = out[6]
    return next_nodes, next_edges, next_globals


# speedup vs baseline: 1.0416x; 1.0416x over previous
import jax
import jax.numpy as jnp
from jax import lax
from jax.experimental import pallas as pl
from jax.experimental.pallas import tpu as pltpu
from jax.experimental.pallas import tpu_sc as plsc

H = 16
DT = 0.01

NW1, NB1, NW2, NB2 = 0, 1, 2, 18
CW1, CB1, CW2, CB2 = 19, 20, 21, 37
LW1, LB1, LW2, LB2 = 38, 39, 40, 56
P1A, P1B, P1C, PB1, P2, PB2 = 57, 73, 89, 105, 106, 122
Q1A, Q1B, QB1, Q2, QB2 = 123, 139, 155, 156, 172
D1, DB1, D2, DB2, D3 = 173, 189, 190, 206, 207
C1, CDB1, C2, CDB2, C3 = 208, 224, 225, 241, 242
L1, LDB1, L2, LDB2, L3 = 243, 259, 260, 276, 277
MISC = 278
NROWS = 279


def _bf16r(v):
    u = lax.bitcast_convert_type(v, jnp.uint32)
    r = (u + jnp.uint32(0x7FFF) + ((u >> jnp.uint32(16)) & jnp.uint32(1))) \
        & jnp.uint32(0xFFFF0000)
    return lax.bitcast_convert_type(r, jnp.float32)


def _forward(p_v):
    zero = jnp.zeros((H,), jnp.float32)

    def mm(specs):
        accs = [[init, zero] for (init, _) in specs]
        rcache = {}
        for (_, pairs) in specs:
            for (av, _wb) in pairs:
                if id(av) not in rcache:
                    rcache[id(av)] = _bf16r(av)
        for k in range(H):
            wcache = {}
            scache = {}
            for j, (_, pairs) in enumerate(specs):
                for (av, wb) in pairs:
                    if wb not in wcache:
                        wcache[wb] = p_v[wb + k, :]
                    ai = id(av)
                    if ai not in scache:
                        scache[ai] = rcache[ai][k]
                    accs[j][k % 2] = accs[j][k % 2] + scache[ai] * wcache[wb]
        return [a[0] + a[1] for a in accs]

    misc = p_v[MISC, :]
    x = [misc[3], misc[4], misc[5]]
    nval = [misc[6], misc[7], misc[8]]

    nw1, nb1 = p_v[NW1, :], p_v[NB1, :]
    an = [jnp.maximum(nval[v] * nw1 + nb1, 0.0) for v in range(3)]

    cw1, cb1 = p_v[CW1, :], p_v[CB1, :]
    lw1, lb1 = p_v[LW1, :], p_v[LB1, :]
    ew1 = [cw1, lw1, cw1]
    eb1 = [cb1, lb1, cb1]
    ae, tae = [], []
    for i in range(3):
        u = x[i] * ew1[i] + eb1[i]
        ae.append(jnp.maximum(u, 0.0))
        tae.append(jnp.where(u > 0, ew1[i], 0.0))

    nb2, cb2, lb2 = p_v[NB2, :], p_v[CB2, :], p_v[LB2, :]
    ew2 = [CW2, LW2, CW2]
    eb2 = [cb2, lb2, cb2]
    res = mm(
        [(nb2, [(an[v], NW2)]) for v in range(3)]
        + [(eb2[i], [(ae[i], ew2[i])]) for i in range(3)]
        + [(zero, [(tae[i], ew2[i])]) for i in range(3)]
    )
    hn0 = res[0:3]
    he0 = res[3:6]
    t0 = res[6:9]

    pb1 = p_v[PB1, :]
    snd = (0, 1, 0)
    rcv = (1, 2, 2)
    res = mm(
        [(pb1, [(he0[i], P1A), (hn0[snd[i]], P1B), (hn0[rcv[i]], P1C)])
         for i in range(3)]
        + [(zero, [(t0[i], P1A)]) for i in range(3)]
    )
    wp, twp = [], []
    for i in range(3):
        u = res[i]
        wp.append(jnp.maximum(u, 0.0))
        twp.append(jnp.where(u > 0, res[3 + i], 0.0))

    pb2 = p_v[PB2, :]
    res = mm(
        [(he0[i] + pb2, [(wp[i], P2)]) for i in range(3)]
        + [(t0[i], [(twp[i], P2)]) for i in range(3)]
    )
    he1 = res[0:3]
    t1 = res[3:6]

    agg = [he1[0], he1[1] + he1[2]]

    qb1 = p_v[QB1, :]
    res = mm([(qb1, [(hn0[1 + v], Q1A), (agg[v], Q1B)]) for v in range(2)])
    nwr = [jnp.maximum(res[v], 0.0) for v in range(2)]

    qb2 = p_v[QB2, :]
    hn1 = mm([(hn0[1 + v] + qb2, [(nwr[v], Q2)]) for v in range(2)])

    db1 = p_v[DB1, :]
    cdb1, ldb1 = p_v[CDB1, :], p_v[LDB1, :]
    e1w = [C1, L1, C1]
    e1b = [cdb1, ldb1, cdb1]
    res = mm(
        [(db1, [(hn1[v], D1)]) for v in range(2)]
        + [(e1b[i], [(he1[i], e1w[i])]) for i in range(3)]
        + [(zero, [(t1[i], e1w[i])]) for i in range(3)]
    )
    z1 = [jnp.maximum(res[v], 0.0) for v in range(2)]
    y1, ty1 = [], []
    for i in range(3):
        u = res[2 + i]
        y1.append(jnp.maximum(u, 0.0))
        ty1.append(jnp.where(u > 0, res[5 + i], 0.0))

    db2 = p_v[DB2, :]
    cdb2, ldb2 = p_v[CDB2, :], p_v[LDB2, :]
    e2w = [C2, L2, C2]
    e2b = [cdb2, ldb2, cdb2]
    res = mm(
        [(db2, [(z1[v], D2)]) for v in range(2)]
        + [(e2b[i], [(y1[i], e2w[i])]) for i in range(3)]
        + [(zero, [(ty1[i], e2w[i])]) for i in range(3)]
    )
    z2 = [jnp.maximum(res[v], 0.0) for v in range(2)]
    y2 = [jnp.maximum(res[2 + i], 0.0) for i in range(3)]
    ty2 = [jnp.where(res[2 + i] > 0, res[5 + i], 0.0) for i in range(3)]

    d3 = p_v[D3, :]
    e3w = [p_v[C3, :], p_v[L3, :], p_v[C3, :]]
    lane = lax.iota(jnp.int32, 16)
    dnums = lax.GatherDimensionNumbers(
        offset_dims=(), collapsed_slice_dims=(0,), start_index_map=(0,))

    z2r = [_bf16r(z2[v]) for v in range(2)]
    y2r = [_bf16r(y2[i]) for i in range(3)]
    ty2r = [_bf16r(ty2[i]) for i in range(3)]
    prods = [
        z2r[0] * d3, z2r[1] * d3,
        ty2r[0] * e3w[0], ty2r[1] * e3w[1], ty2r[2] * e3w[2],
        y2r[0] * e3w[0] + y2r[1] * e3w[1] + y2r[2] * e3w[2],
    ]
    for sh in (1, 2, 4, 8):
        idx = (lane ^ sh)[:, None]
        prods = [
            v + lax.gather(v, idx, dnums, slice_sizes=(1,),
                           mode=lax.GatherScatterMode.PROMISE_IN_BOUNDS)
            for v in prods
        ]
    db3n = misc[0]
    decn = [prods[0] + db3n, prods[1] + db3n]
    dh = prods[2:5]
    ham = prods[5] + misc[1] + misc[2] + misc[1]

    c2 = misc[11]
    nx0 = x[0] + DT * dh[1]
    nx1 = x[1] + DT * (-dh[0] + dh[2])
    nx2 = x[2] + DT * (-dh[1] - c2)
    return (
        jnp.where(lane == 1, decn[0], 0.0)
        + jnp.where(lane == 2, decn[1], 0.0)
        + jnp.where(lane == 3, nx0, 0.0)
        + jnp.where(lane == 4, nx1, 0.0)
        + jnp.where(lane == 5, nx2, 0.0)
        + jnp.where(lane == 6, ham, 0.0)
    )


def _body(p_hbm, out_hbm, p_v, o_v):
    c = lax.axis_index("c")
    s = lax.axis_index("s")

    @pl.when(jnp.logical_and(c == 0, s == 0))
    def _():
        pltpu.sync_copy(p_hbm, p_v)
        o_v[...] = _forward(p_v)
        pltpu.sync_copy(o_v, out_hbm)


def _pack(nodes, edges, control, enc_node, enc_edgeC, enc_edgeL, proc_edge, proc_node, dec_node, dec_edgeC, dec_edgeL):
    def r(w):
        return _bf16r(w).reshape(-1)

    parts = []
    for p in (enc_node, enc_edgeC, enc_edgeL):
        parts.extend([p[0].reshape(-1), p[1], r(p[2]), p[3]])
    for p in (proc_edge, proc_node):
        parts.extend([r(p[0]), p[1], r(p[2]), p[3]])
    for p in (dec_node, dec_edgeC, dec_edgeL):
        parts.extend([r(p[0]), p[1], r(p[2]), p[3], r(p[4])])
    parts.extend([
        dec_node[5], dec_edgeC[5], dec_edgeL[5],
        edges.reshape(-1), nodes.reshape(-1), control,
        jnp.zeros((4,), jnp.float32),
    ])
    return jnp.concatenate(parts).reshape(NROWS, H)


def kernel(nodes, edges, control, enc_node, enc_edgeC, enc_edgeL, proc_edge, proc_node, dec_node, dec_edgeC, dec_edgeL):
    packed = _pack(nodes, edges, control, enc_node, enc_edgeC, enc_edgeL,
                   proc_edge, proc_node, dec_node, dec_edgeC, dec_edgeL)

    mesh = plsc.VectorSubcoreMesh(core_axis_name="c", subcore_axis_name="s",
                                  num_cores=1)
    run = pl.kernel(
        _body,
        out_type=jax.ShapeDtypeStruct((H,), jnp.float32),
        mesh=mesh,
        scratch_types=[
            pltpu.VMEM((NROWS, H), jnp.float32),
            pltpu.VMEM((H,), jnp.float32),
        ],
    )
    out = run(packed)

    next_nodes = out[0:3].reshape(3, 1)
    next_edges = out[3:6].reshape(3, 1)
    next_globals = out[6]
    return next_nodes, next_edges, next_globals
